# Initial kernel scaffold; baseline (speedup 1.0000x reference)
import jax
import jax.numpy as jnp
from jax.experimental import pallas as pl


def kernel(x, edge_index, batch, W0, b0, W1, b1, W2, b2, W3, b3, Wc1, bc1, Wc2, bc2, Wm1, bm1, Wm2, bm2):
    raise NotImplementedError("write your pallas kernel here")


# speedup vs baseline: 14.8976x; 14.8976x over previous
import functools

import jax
import jax.numpy as jnp
import numpy as np
from jax import lax
from jax.experimental import pallas as pl
from jax.experimental.pallas import tpu as pltpu
from jax.experimental.pallas import tpu_sc as plsc

N = 10000
E = 160000
HID = 256
B = 64
K = 30

NP = 10240
EP = 163840
NC = 2
NS = 16
STRIPE = NP // NS
NEG = np.float32(-1e30)

_mesh = plsc.VectorSubcoreMesh(core_axis_name="c", subcore_axis_name="s")
_sc_params = pltpu.CompilerParams(needs_layout_passes=False)


def _scalar_agg_body(table, srcf, dst2d, out, tbl_v, srcv, dstv, valv, zv, acc):
    c = lax.axis_index("c")
    s = lax.axis_index("s")
    w = c * NS + s
    epw = EP // (NC * NS)
    base = w * epw
    pltpu.sync_copy(table, tbl_v)
    pltpu.sync_copy(srcf.at[pl.ds(base, epw)], srcv)
    pltpu.sync_copy(dst2d.at[pl.ds(w * (epw // 128), epw // 128)], dstv)

    def zero_body(i, _):
        zv[pl.ds(pl.multiple_of(i * 16, 16), 16)] = jnp.zeros((16,), jnp.float32)
        return 0
    lax.fori_loop(0, STRIPE // 16, zero_body, 0)
    pltpu.sync_copy(zv, acc.at[pl.ds(s * STRIPE, STRIPE)])
    plsc.subcore_barrier()

    def gather_body(g, _):
        off = pl.multiple_of(g * 16, 16)
        iv = srcv[pl.ds(off, 16)]
        v = plsc.load_gather(tbl_v, [iv])
        valv[pl.ds(off, 16)] = v
        return 0
    lax.fori_loop(0, epw // 16, gather_body, 0)

    for j in range(epw // 128):
        pltpu.sync_copy(valv.at[pl.ds(j * 128, 128)], acc.at[dstv.at[j]],
                        add=True)
    plsc.subcore_barrier()
    pltpu.sync_copy(acc.at[pl.ds(s * STRIPE, STRIPE)],
                    out.at[c, pl.ds(s * STRIPE, STRIPE)])


_scalar_agg = functools.partial(
    pl.kernel, _scalar_agg_body, mesh=_mesh, compiler_params=_sc_params,
    out_type=jax.ShapeDtypeStruct((NC, NP), jnp.float32),
    scratch_types=[
        pltpu.VMEM((NP,), jnp.float32),
        pltpu.VMEM((EP // (NC * NS),), jnp.int32),
        pltpu.VMEM((EP // (NC * NS) // 128, 128), jnp.int32),
        pltpu.VMEM((EP // (NC * NS),), jnp.float32),
        pltpu.VMEM((STRIPE,), jnp.float32),
        pltpu.VMEM_SHARED((NP,), jnp.float32),
    ],
)()


def _row_agg_body(hs2, srcf, dst2d, out, srcv, dstv, g0, g1, acc, sem0, sem1):
    c = lax.axis_index("c")
    s = lax.axis_index("s")
    eps = EP // NS
    base = s * eps
    stripe = pl.ds(s * STRIPE, STRIPE)
    pltpu.sync_copy(hs2.at[pl.ds(c * NP + s * STRIPE, STRIPE)], acc.at[stripe])
    pltpu.sync_copy(dst2d.at[pl.ds(s * (eps // 128), eps // 128)], dstv)
    plsc.subcore_barrier()

    off = c * NP
    nsec = 4
    spc = eps // nsec
    cps = spc // 128
    bufs = (g0, g1)
    sems = (sem0, sem1)
    descs = [None, None]

    for sec in range(nsec):
        pltpu.sync_copy(srcf.at[pl.ds(base + sec * spc, spc)], srcv)

        def off_body(i, _):
            d = pl.ds(pl.multiple_of(i * 16, 16), 16)
            srcv[d] = srcv[d] + off
            return 0
        lax.fori_loop(0, spc // 16, off_body, 0)

        def start(j):
            iv = srcv.at[pl.ds(j * 128, 128)]
            descs[j % 2] = pltpu.async_copy(hs2.at[iv], bufs[j % 2],
                                            sems[j % 2])

        start(0)
        for j in range(cps):
            if j + 1 < cps:
                start(j + 1)
            descs[j % 2].wait()
            pltpu.sync_copy(bufs[j % 2], acc.at[dstv.at[sec * cps + j]],
                            add=True)
    plsc.subcore_barrier()
    pltpu.sync_copy(acc.at[stripe], out.at[c, stripe])


_row_agg = functools.partial(
    pl.kernel, _row_agg_body, mesh=_mesh, compiler_params=_sc_params,
    out_type=jax.ShapeDtypeStruct((NC, NP, 128), jnp.float32),
    scratch_types=[
        pltpu.VMEM((EP // NS // 4,), jnp.int32),
        pltpu.VMEM((EP // NS // 128, 128), jnp.int32),
        pltpu.VMEM((128, 128), jnp.float32),
        pltpu.VMEM((128, 128), jnp.float32),
        pltpu.VMEM_SHARED((NP, 128), jnp.float32),
        pltpu.SemaphoreType.DMA,
        pltpu.SemaphoreType.DMA,
    ],
)()


def _topk_body(x4v, starts, counts, proj, pooled, vbuf, sbuf, cbuf, ibuf,
               prow, sem):
    c = lax.axis_index("c")
    s = lax.axis_index("s")
    w = c * NS + s
    pltpu.sync_copy(x4v, vbuf)
    pltpu.sync_copy(starts, sbuf)
    pltpu.sync_copy(counts, cbuf)

    sent = jnp.full((16,), N, jnp.int32)
    for q in range(4):
        ibuf[pl.ds(q * 16, 16)] = sent
    iota = lax.iota(jnp.int32, 16)
    lane0 = iota == 0

    def _extract(ref, g):
        off = pl.multiple_of(lax.div(g, 16) * 16, 16)
        v = ref[pl.ds(off, 16)]
        return jnp.sum(jnp.where(iota == lax.rem(g, 16), v, 0))

    for gi in range(2):
        g = w * 2 + gi
        st = _extract(sbuf, g)
        cn = _extract(cbuf, g)
        lo = lax.div(st, 16)
        hi = lax.div(st + cn + 15, 16)
        for k in range(K):
            def max_body(i, m16):
                off = pl.multiple_of(i * 16, 16)
                v = vbuf[pl.ds(off, 16)]
                pos = iota + i * 16
                msk = (pos >= st) & (pos < st + cn)
                return jnp.maximum(m16, jnp.where(msk, v, NEG))
            m16 = lax.fori_loop(lo, hi, max_body,
                                jnp.full((16,), NEG, jnp.float32))
            mx = jnp.max(m16)

            def min_body(i, i16):
                off = pl.multiple_of(i * 16, 16)
                v = vbuf[pl.ds(off, 16)]
                pos = iota + i * 16
                msk = (pos >= st) & (pos < st + cn) & (v >= mx)
                return jnp.minimum(i16, jnp.where(msk, pos, jnp.int32(NP)))
            i16 = lax.fori_loop(lo, hi, min_body,
                                jnp.full((16,), NP, jnp.int32))
            im = jnp.min(i16)
            isel = jnp.where(mx > jnp.float32(-1e29), im, jnp.int32(N))
            plsc.store_scatter(ibuf, [jnp.full((16,), gi * K + k, jnp.int32)],
                               jnp.full((16,), 1, jnp.int32) * isel, mask=lane0)
            plsc.store_scatter(vbuf, [jnp.full((16,), 1, jnp.int32) * isel],
                               jnp.full((16,), NEG, jnp.float32), mask=lane0)
    pltpu.async_copy(proj.at[ibuf], prow, sem).wait()
    pltpu.sync_copy(prow, pooled.at[pl.ds(w * 64, 64)])


_topk = functools.partial(
    pl.kernel, _topk_body, mesh=_mesh, compiler_params=_sc_params,
    out_type=jax.ShapeDtypeStruct((NC * NS * 64, 128), jnp.float32),
    scratch_types=[
        pltpu.VMEM((NP,), jnp.float32),
        pltpu.VMEM((B,), jnp.int32),
        pltpu.VMEM((B,), jnp.int32),
        pltpu.VMEM((64,), jnp.int32),
        pltpu.VMEM((64, 128), jnp.float32),
        pltpu.SemaphoreType.DMA,
    ],
)()


def _dinv_block(degp):
    deg = degp[0] + degp[1] + 1.0
    return (1.0 / jnp.sqrt(deg))[:, None]


def _t1_body(x_ref, w_ref, degp_ref, hs_ref):
    dinv = _dinv_block(degp_ref[...])
    h = jnp.dot(x_ref[...], w_ref[...], preferred_element_type=jnp.float32)
    hs = h * dinv
    hs_ref[0] = hs[:, :128]
    hs_ref[1] = hs[:, 128:]


def _t2_body(t_ref, degp_ref, b_ref, w_ref, x_ref, hs_ref):
    tb = t_ref[...]
    t = jnp.concatenate([tb[0], tb[1]], axis=1)
    dinv = _dinv_block(degp_ref[...])
    xl = jnp.tanh(t * dinv + b_ref[...])
    x_ref[...] = xl
    h = jnp.dot(xl, w_ref[...], preferred_element_type=jnp.float32)
    hs = h * dinv
    hs_ref[0] = hs[:, :128]
    hs_ref[1] = hs[:, 128:]


def _t3_body(t_ref, degp_ref, b_ref, w_ref, x_ref, hs4_ref):
    tb = t_ref[...]
    t = jnp.concatenate([tb[0], tb[1]], axis=1)
    dinv = _dinv_block(degp_ref[...])
    xl = jnp.tanh(t * dinv + b_ref[...])
    x_ref[...] = xl
    h = jnp.dot(xl, w_ref[...], preferred_element_type=jnp.float32)
    hs4_ref[...] = h * dinv


def _t4_body(i, t4p_ref, hs4_ref, degp_ref, b3_ref, x1_ref, x2_ref, x3_ref,
             ba_ref, wa_ref, wb_ref, wc_ref, wl_ref,
             x4_ref, proj_ref, cnt_ref, str_ref):
    dinv = _dinv_block(degp_ref[...])
    t4 = t4p_ref[...]
    t4s = (t4[0] + t4[1])[:, None] + hs4_ref[...]
    x4 = jnp.tanh(t4s * dinv + b3_ref[...])
    x4_ref[...] = x4
    proj = (jnp.dot(x1_ref[...], wa_ref[...], preferred_element_type=jnp.float32)
            + jnp.dot(x2_ref[...], wb_ref[...], preferred_element_type=jnp.float32)
            + jnp.dot(x3_ref[...], wc_ref[...], preferred_element_type=jnp.float32)
            + x4 * wl_ref[...])
    rows = i * 512 + lax.broadcasted_iota(jnp.int32, (512, 1), 0)
    proj128 = jnp.concatenate([proj, jnp.zeros((512, 112), jnp.float32)],
                              axis=1)
    proj_ref[...] = jnp.where(rows < N, proj128, 0.0)
    bb = ba_ref[...]
    eq = (bb == lax.broadcasted_iota(jnp.int32, (1, B), 1)).astype(jnp.int32)
    @pl.when(i == 0)
    def _():
        cnt_ref[...] = jnp.zeros((1, B), jnp.int32)
        str_ref[...] = jnp.zeros((1, B), jnp.int32)
    cnt_ref[...] += jnp.sum(eq, axis=0, keepdims=True)
    @pl.when(i == (NP // 512) - 1)
    def _():
        cf = cnt_ref[...].astype(jnp.float32)
        r0 = lax.broadcasted_iota(jnp.int32, (B, B), 0)
        r1 = lax.broadcasted_iota(jnp.int32, (B, B), 1)
        tri = (r0 < r1).astype(jnp.float32)
        str_ref[...] = jnp.dot(cf, tri,
                               preferred_element_type=jnp.float32).astype(jnp.int32)


def _t4_wrap(t4p, hs4, degp, b3, x1, x2, x3, ba, wa, wb, wc, wl):
    grid = NP // 512
    return pl.pallas_call(
        lambda *refs: _t4_body(pl.program_id(0), *refs),
        grid=(grid,),
        in_specs=[
            pl.BlockSpec((2, 512), lambda i: (0, i)),
            pl.BlockSpec((512, 1), lambda i: (i, 0)),
            pl.BlockSpec((2, 512), lambda i: (0, i)),
            pl.BlockSpec((1, 1), lambda i: (0, 0)),
            pl.BlockSpec((512, HID), lambda i: (i, 0)),
            pl.BlockSpec((512, HID), lambda i: (i, 0)),
            pl.BlockSpec((512, HID), lambda i: (i, 0)),
            pl.BlockSpec((512, 1), lambda i: (i, 0)),
            pl.BlockSpec((HID, 16), lambda i: (0, 0)),
            pl.BlockSpec((HID, 16), lambda i: (0, 0)),
            pl.BlockSpec((HID, 16), lambda i: (0, 0)),
            pl.BlockSpec((1, 16), lambda i: (0, 0)),
        ],
        out_specs=[
            pl.BlockSpec((512, 1), lambda i: (i, 0)),
            pl.BlockSpec((512, 128), lambda i: (i, 0)),
            pl.BlockSpec((1, B), lambda i: (0, 0)),
            pl.BlockSpec((1, B), lambda i: (0, 0)),
        ],
        out_shape=[
            jax.ShapeDtypeStruct((NP, 1), jnp.float32),
            jax.ShapeDtypeStruct((NP, 128), jnp.float32),
            jax.ShapeDtypeStruct((1, B), jnp.int32),
            jax.ShapeDtypeStruct((1, B), jnp.int32),
        ],
    )(t4p, hs4, degp, b3, x1, x2, x3, ba, wa, wb, wc, wl)


def _t5_body(pp_ref, bc1_ref, e_ref, o_ref, wcv_ref, bc2_ref, wm1_ref,
             bm1_ref, wm2_ref, bm2_ref, out_ref):
    h1 = jax.nn.relu(pp_ref[...] + bc1_ref[...])
    me = jnp.dot(h1, e_ref[...], preferred_element_type=jnp.float32)
    mo = jnp.dot(h1, o_ref[...], preferred_element_type=jnp.float32)
    m = jnp.maximum(me, mo)
    cc = jax.nn.relu(jnp.dot(m, wcv_ref[...], preferred_element_type=jnp.float32)
                     + bc2_ref[...])
    h2 = jax.nn.relu(jnp.dot(cc, wm1_ref[...], preferred_element_type=jnp.float32)
                     + bm1_ref[...])
    out_ref[...] = (jnp.dot(h2, wm2_ref[...], preferred_element_type=jnp.float32)
                    + bm2_ref[...])


_ESEL = np.zeros((480, 240), np.float32)
_OSEL = np.zeros((480, 240), np.float32)
for _j in range(15):
    for _ch in range(16):
        _ESEL[(2 * _j) * 16 + _ch, _j * 16 + _ch] = 1.0
        _OSEL[(2 * _j + 1) * 16 + _ch, _j * 16 + _ch] = 1.0


def kernel(x, edge_index, batch, W0, b0, W1, b1, W2, b2, W3, b3, Wc1, bc1,
           Wc2, bc2, Wm1, bm1, Wm2, bm2):
    f32 = jnp.float32
    pad = N + (jnp.arange(EP - E, dtype=jnp.int32) % 16)
    srcf = jnp.concatenate([edge_index[0], pad])
    dstf = jnp.concatenate([edge_index[1], pad])
    dst2d = dstf.reshape(EP // 128, 128)
    x_pad = jnp.zeros((NP, HID), f32).at[:N].set(x)
    batch_pad = jnp.full((NP, 1), B, jnp.int32).at[:N, 0].set(batch)

    degp = _scalar_agg(jnp.ones((NP,), f32), srcf, dst2d)

    grid = NP // 512
    hs_spec = pl.BlockSpec((2, 512, 128), lambda i: (0, i, 0))
    t_spec = pl.BlockSpec((2, 512, 128), lambda i: (0, i, 0))
    degp_spec = pl.BlockSpec((2, 512), lambda i: (0, i))
    xs_spec = pl.BlockSpec((512, HID), lambda i: (i, 0))
    w_spec = pl.BlockSpec((HID, HID), lambda i: (0, 0))
    b_spec = pl.BlockSpec((1, HID), lambda i: (0, 0))
    hs_shape = jax.ShapeDtypeStruct((2, NP, 128), f32)

    hs = pl.pallas_call(
        _t1_body, grid=(grid,),
        in_specs=[xs_spec, w_spec, degp_spec],
        out_specs=hs_spec, out_shape=hs_shape,
    )(x_pad, W0, degp)

    def tc_layer(t_cat, b_prev, W_next):
        return pl.pallas_call(
            _t2_body, grid=(grid,),
            in_specs=[t_spec, degp_spec, b_spec, w_spec],
            out_specs=[xs_spec, hs_spec],
            out_shape=[jax.ShapeDtypeStruct((NP, HID), f32), hs_shape],
        )(t_cat, degp, b_prev, W_next)

    t = _row_agg(hs.reshape(2 * NP, 128), srcf, dst2d)
    x1, hs = tc_layer(t, b0.reshape(1, HID), W1)
    t = _row_agg(hs.reshape(2 * NP, 128), srcf, dst2d)
    x2, hs = tc_layer(t, b1.reshape(1, HID), W2)
    t = _row_agg(hs.reshape(2 * NP, 128), srcf, dst2d)

    x3, hs4 = pl.pallas_call(
        _t3_body, grid=(grid,),
        in_specs=[t_spec, degp_spec, b_spec,
                  pl.BlockSpec((HID, 1), lambda i: (0, 0))],
        out_specs=[xs_spec, pl.BlockSpec((512, 1), lambda i: (i, 0))],
        out_shape=[jax.ShapeDtypeStruct((NP, HID), f32),
                   jax.ShapeDtypeStruct((NP, 1), f32)],
    )(t, degp, b2.reshape(1, HID), W3)
    t4p = _scalar_agg(hs4.reshape(NP), srcf, dst2d)

    x4v, proj, counts, starts = _t4_wrap(
        t4p, hs4, degp, b3.reshape(1, 1), x1, x2, x3, batch_pad,
        Wc1[:, 0:256].T, Wc1[:, 256:512].T, Wc1[:, 512:768].T,
        Wc1[:, 768].reshape(1, 16))

    pooled = _topk(x4v.reshape(NP), starts.reshape(B), counts.reshape(B), proj)

    pp = pooled.reshape(32, 64, 128)[:, :60, :16].reshape(B, K * 16)
    bc1t = jnp.tile(bc1, K)[None]
    bc2t = jnp.tile(bc2, 11)[None]
    wconv = jnp.zeros((240, 352), f32)
    for p in range(11):
        for tt in range(5):
            wconv = wconv.at[(p + tt) * 16:(p + tt + 1) * 16,
                             p * 32:(p + 1) * 32].add(Wc2[:, :, tt].T)
    wm1p = Wm1.reshape(32, 11, 128).transpose(1, 0, 2).reshape(352, 128)

    full = lambda r, c: pl.BlockSpec((r, c), lambda: (0, 0))
    out = pl.pallas_call(
        _t5_body,
        in_specs=[full(B, 480), full(1, 480), full(480, 240), full(480, 240),
                  full(240, 352), full(1, 352), full(352, 128), full(1, 128),
                  full(128, 1), full(1, 1)],
        out_specs=full(B, 1),
        out_shape=jax.ShapeDtypeStruct((B, 1), f32),
    )(pp, bc1t, _ESEL, _OSEL, wconv, bc2t, wm1p, bm1.reshape(1, 128),
      Wm2, bm2.reshape(1, 1))
    return out
